# baseline (speedup 1.0000x reference)
import functools

import jax
import jax.numpy as jnp
from jax import lax
from jax.experimental import pallas as pl
from jax.experimental.pallas import tpu as pltpu
from jax.experimental.pallas import tpu_sc as plsc

EMBED = 256
CHUNK = 128


@functools.lru_cache(maxsize=None)
def _make_lookup(n_rows: int):
    info = plsc.get_sparse_core_info()
    nw = info.num_cores * info.num_subcores
    assert n_rows % (8 * nw) == 0
    per_w = n_rows // nw
    chunk = min(CHUNK, per_w)
    n_chunks = per_w // chunk
    mesh = plsc.VectorSubcoreMesh(core_axis_name="c", subcore_axis_name="s")

    @functools.partial(
        pl.kernel,
        mesh=mesh,
        out_type=jax.ShapeDtypeStruct((n_rows, EMBED), jnp.float32),
        scratch_types=[
            pltpu.VMEM((per_w,), jnp.int32),
            pltpu.VMEM((2, chunk, EMBED), jnp.float32),
            pltpu.SemaphoreType.DMA,
            pltpu.SemaphoreType.DMA,
            pltpu.SemaphoreType.DMA,
        ],
    )
    def lookup(table_hbm, idx_hbm, out_hbm, idx_v, rows_v, gsem0, gsem1, ssem):
        wid = lax.axis_index("s") * info.num_cores + lax.axis_index("c")
        base = wid * per_w
        pltpu.sync_copy(idx_hbm.at[pl.ds(base, per_w)], idx_v)

        gsems = (gsem0, gsem1)
        gathers = [None] * n_chunks
        scatters = [None] * n_chunks

        def start_gather(i):
            gathers[i] = pltpu.async_copy(
                table_hbm.at[idx_v.at[pl.ds(i * chunk, chunk)]],
                rows_v.at[i % 2],
                gsems[i % 2],
            )

        def start_scatter(i):
            off = pl.multiple_of(base + i * chunk, 8)
            scatters[i] = pltpu.async_copy(
                rows_v.at[i % 2], out_hbm.at[pl.ds(off, chunk)], ssem
            )

        start_gather(0)
        for i in range(n_chunks):
            if i + 1 < n_chunks:
                if i >= 1:
                    scatters[i - 1].wait()
                start_gather(i + 1)
            gathers[i].wait()
            start_scatter(i)
        if n_chunks >= 2:
            scatters[n_chunks - 2].wait()
        scatters[n_chunks - 1].wait()

    return lookup


def kernel(x, table):
    b, s = x.shape
    n = b * s
    idx = x.reshape(n).astype(jnp.int32)
    out = _make_lookup(n)(table.astype(jnp.float32), idx)
    return out.reshape(b, s, EMBED)


# speedup vs baseline: 3.6560x; 3.6560x over previous
import functools

import jax
import jax.numpy as jnp
from jax import lax
from jax.experimental import pallas as pl
from jax.experimental.pallas import tpu as pltpu
from jax.experimental.pallas import tpu_sc as plsc

EMBED = 256
CHUNK = 128


@functools.lru_cache(maxsize=None)
def _make_lookup(n_rows: int, vocab: int):
    info = plsc.get_sparse_core_info()
    nw = info.num_cores * info.num_subcores
    assert n_rows % (8 * nw) == 0
    per_w = n_rows // nw
    chunk = min(CHUNK, per_w)
    n_chunks = per_w // chunk
    mesh = plsc.VectorSubcoreMesh(core_axis_name="c", subcore_axis_name="s")

    @functools.partial(
        pl.kernel,
        mesh=mesh,
        out_type=jax.ShapeDtypeStruct((n_rows, EMBED), jnp.float32),
        scratch_types=[
            pltpu.VMEM((vocab, EMBED), jnp.float32),
            pltpu.VMEM((per_w,), jnp.int32),
            pltpu.VMEM((2, chunk, EMBED), jnp.float32),
            pltpu.SemaphoreType.DMA,
        ],
    )
    def lookup(table_hbm, idx_hbm, out_hbm, table_v, idx_v, rows_v, ssem):
        wid = lax.axis_index("s") * info.num_cores + lax.axis_index("c")
        base = wid * per_w
        pltpu.sync_copy(table_hbm, table_v)
        pltpu.sync_copy(idx_hbm.at[pl.ds(base, per_w)], idx_v)

        scatters = [None] * n_chunks

        def fill_chunk(i):
            buf = i % 2

            def group_body(g, carry):
                iv = idx_v[pl.ds(i * chunk + g * 16, 16)]
                for r in range(16):
                    s = iv[r]
                    for j in range(EMBED // 16):
                        rows_v[buf, g * 16 + r, pl.ds(j * 16, 16)] = (
                            table_v[s, pl.ds(j * 16, 16)]
                        )
                return carry

            lax.fori_loop(0, chunk // 16, group_body, 0)

        for i in range(n_chunks):
            if i >= 2:
                scatters[i - 2].wait()
            fill_chunk(i)
            off = pl.multiple_of(base + i * chunk, 8)
            scatters[i] = pltpu.async_copy(
                rows_v.at[i % 2], out_hbm.at[pl.ds(off, chunk)], ssem
            )
        scatters[n_chunks - 2].wait()
        scatters[n_chunks - 1].wait()

    return lookup


def kernel(x, table):
    b, s = x.shape
    n = b * s
    idx = x.reshape(n).astype(jnp.int32)
    out = _make_lookup(n, table.shape[0])(table.astype(jnp.float32), idx)
    return out.reshape(b, s, EMBED)
